# baseline (speedup 1.0000x reference)
import functools

import jax
import jax.numpy as jnp
from jax import lax
from jax.experimental import pallas as pl
from jax.experimental.pallas import tpu as pltpu
from jax.experimental.pallas import tpu_sc as plsc

B = 4096
L = 200
EMB = 64
NW = 32
BPW = B // NW
NCH = 2
CH = L // NCH
QV = EMB // 16


def _body(x_hbm, w_hbm, out_hbm, idx_v, rows_v, out_v, sem0, sem1):
    c = lax.axis_index("c")
    s = lax.axis_index("s")
    wid = s * 2 + c
    base = wid * BPW

    pltpu.sync_copy(x_hbm.at[pl.ds(base, BPW)], idx_v)

    sems = (sem0, sem1)

    def start(b, slot):
        for j in range(NCH):
            pltpu.async_copy(
                w_hbm.at[idx_v.at[b, j]],
                rows_v.at[slot, pl.ds(j * CH, CH)],
                sems[slot],
            )

    def wait(slot):
        for j in range(NCH):
            pltpu.make_async_copy(
                w_hbm.at[idx_v.at[0, j]],
                rows_v.at[slot, pl.ds(j * CH, CH)],
                sems[slot],
            ).wait()

    start(0, 0)
    start(1, 1)

    def accum(slot, b):
        def inner(l, acc):
            return tuple(
                acc[q] + rows_v[slot, l, pl.ds(16 * q, 16)] for q in range(QV)
            )
        zero = jnp.zeros((16,), jnp.float32)
        acc = lax.fori_loop(0, L, inner, (zero,) * QV)
        scale = jnp.float32(1.0 / L)
        for q in range(QV):
            out_v[b, pl.ds(16 * q, 16)] = acc[q] * scale

    def outer(g, carry):
        for slot in range(2):
            b = g * 2 + slot
            wait(slot)
            nb = b + 2

            @pl.when(nb < BPW)
            def _():
                start(nb, slot)

            accum(slot, b)
        return carry

    lax.fori_loop(0, BPW // 2, outer, 0)

    pltpu.sync_copy(out_v, out_hbm.at[pl.ds(base, BPW)])


_mesh = plsc.VectorSubcoreMesh(core_axis_name="c", subcore_axis_name="s")

_sc_call = pl.kernel(
    _body,
    mesh=_mesh,
    out_type=jax.ShapeDtypeStruct((B, EMB), jnp.float32),
    scratch_types=[
        pltpu.VMEM((BPW, NCH, CH), jnp.int32),
        pltpu.VMEM((2, L, EMB), jnp.float32),
        pltpu.VMEM((BPW, EMB), jnp.float32),
        pltpu.SemaphoreType.DMA,
        pltpu.SemaphoreType.DMA,
    ],
    compiler_params=pltpu.CompilerParams(use_tc_tiling_on_sc=False),
)


@jax.jit
def _run(x3, w):
    return _sc_call(x3, w)


def kernel(x, sizes, emb_weight):
    del sizes
    x3 = x.reshape(B, NCH, CH)
    return _run(x3, emb_weight)
